# baseline (speedup 1.0000x reference)
import functools

import jax
import jax.numpy as jnp
import numpy as np
from jax import lax
from jax.experimental import pallas as pl
from jax.experimental.pallas import tpu as pltpu
from jax.experimental.pallas import tpu_sc as plsc

N_CORES = 2
N_SUB = 16
N_WORKERS = N_CORES * N_SUB
CH = 112

_MESH = plsc.VectorSubcoreMesh(
    core_axis_name="c", subcore_axis_name="s",
    num_cores=N_CORES, num_subcores=N_SUB)
_SC_PARAMS = pltpu.CompilerParams(needs_layout_passes=False)


def _deg_body(n_pad, rows_pw, dst3d, w3d, zeros1d, degp, deg_sh, dstv, wv):
    c = lax.axis_index("c")
    s = lax.axis_index("s")
    wid = c * N_SUB + s
    per_tile = n_pad // N_SUB
    pltpu.sync_copy(zeros1d.at[pl.ds(s * per_tile, per_tile)],
                    deg_sh.at[pl.ds(s * per_tile, per_tile)])
    plsc.subcore_barrier()
    r0 = wid * rows_pw
    pltpu.sync_copy(dst3d.at[pl.ds(r0, rows_pw)], dstv)
    pltpu.sync_copy(w3d.at[pl.ds(r0, rows_pw)], wv)

    def chunk(i, _):
        pltpu.sync_copy(wv.at[i, 0], deg_sh.at[dstv.at[i, 0]], add=True)
        return 0

    lax.fori_loop(0, rows_pw, chunk, 0)
    plsc.subcore_barrier()
    pltpu.sync_copy(deg_sh.at[pl.ds(s * per_tile, per_tile)],
                    degp.at[pl.ds(c * n_pad + s * per_tile, per_tile)])


def _conv_body(n, rows_pw, y_hbm, src3d, dst3d, w3d, zeros2d, acc_out,
               acc_sh, rows0, rows1, rows2, sb0, sb1, sb2, db0, db1, db2,
               wb0, wb1, wb2, sg0, sg1, sg2, ss0, ss1, ss2,
               es0, es1, es2, ed0, ed1, ed2, ew0, ew1, ew2):
    c = lax.axis_index("c")
    s = lax.axis_index("s")
    wid = c * N_SUB + s
    per_tile = n // N_SUB
    pltpu.sync_copy(zeros2d.at[pl.ds(s * per_tile, per_tile)],
                    acc_sh.at[pl.ds(s * per_tile, per_tile)])
    plsc.subcore_barrier()
    r0 = wid * rows_pw

    bufs = (rows0, rows1, rows2)
    srcb = (sb0, sb1, sb2)
    dstb = (db0, db1, db2)
    wb = (wb0, wb1, wb2)
    sg = (sg0, sg1, sg2)
    ss = (ss0, ss1, ss2)
    es = (es0, es1, es2)
    ed = (ed0, ed1, ed2)
    ew = (ew0, ew1, ew2)

    def load_src_w(i, k):
        pltpu.async_copy(src3d.at[r0 + i, 0], srcb[k], es[k])
        pltpu.async_copy(w3d.at[r0 + i, 0], wb[k], ew[k])

    def load_dst(i, k):
        pltpu.async_copy(dst3d.at[r0 + i, 0], dstb[k], ed[k])

    def scale(p):
        @plsc.parallel_loop(0, CH, unroll=8)
        def body(e):
            ws = plsc.load_gather(wb[p], [jnp.full((16,), e, jnp.int32)])
            for f in range(8):
                sl = pl.ds(f * 16, 16)
                bufs[p][e, sl] = bufs[p][e, sl] * ws

    def step(i, p, r):
        @pl.when(jnp.logical_and(i >= 2, i < rows_pw + 2))
        def _():
            pltpu.make_async_copy(bufs[r], acc_sh.at[dstb[r]],
                                  ss[r]).wait()

        @pl.when(i + 1 < rows_pw)
        def _():
            pltpu.make_async_copy(src3d.at[r0, 0], srcb[r], es[r]).wait()
            pltpu.async_copy(y_hbm.at[srcb[r]], bufs[r], sg[r])

        @pl.when(i + 2 < rows_pw)
        def _():
            load_src_w(i + 2, (p + 2) % 3)

        @pl.when(i + 1 < rows_pw)
        def _():
            load_dst(i + 1, r)

        @pl.when(i < rows_pw)
        def _():
            pltpu.make_async_copy(y_hbm.at[srcb[p]], bufs[p], sg[p]).wait()
            pltpu.make_async_copy(w3d.at[r0, 0], wb[p], ew[p]).wait()
            scale(p)
            pltpu.make_async_copy(dst3d.at[r0, 0], dstb[p], ed[p]).wait()
            pltpu.async_copy(bufs[p], acc_sh.at[dstb[p]], ss[p], add=True)

    load_src_w(0, 0)
    load_dst(0, 0)
    load_src_w(1, 1)
    pltpu.make_async_copy(src3d.at[r0, 0], srcb[0], es[0]).wait()
    pltpu.async_copy(y_hbm.at[srcb[0]], bufs[0], sg[0])

    n_triples = (rows_pw + 2 + 2) // 3

    def triple(t, _):
        i0 = 3 * t
        step(i0, 0, 1)
        step(i0 + 1, 1, 2)
        step(i0 + 2, 2, 0)
        return 0

    lax.fori_loop(0, n_triples, triple, 0)

    plsc.subcore_barrier()
    pltpu.sync_copy(acc_sh.at[pl.ds(s * per_tile, per_tile)],
                    acc_out.at[c, pl.ds(s * per_tile, per_tile)])


def _tc0_body(x, w1, xw_o):
    xw_o[...] = jnp.dot(x[...], w1[...],
                        preferred_element_type=jnp.float32)


def _tc1_body(deg0, deg1, xw, dinv_o, y1_o):
    deg = deg0[...] + deg1[...] + 1.0
    dinv = lax.rsqrt(deg)
    dinv_o[...] = dinv
    y1_o[...] = xw[...] * dinv


def _tc2_body(acc, dinv, xw, b1, w2, gw_o, y2_o):
    dv = dinv[...]
    g = acc[0] + acc[1]
    g = jnp.maximum(dv * g + dv * dv * xw[...] + b1[...], 0.0)
    gw = jnp.dot(g, w2[...], preferred_element_type=jnp.float32)
    gw_o[...] = gw
    y2_o[...] = gw * dv


def _tc3_body(acc, dinv, gw, b2, h, wua, wub, bu, wra, wrb, br, wca, wcb,
              bc, out_o):
    dv = dinv[...]
    g = acc[0] + acc[1]
    xg = jax.nn.sigmoid(dv * g + dv * dv * gw[...] + b2[...])
    hh = h[...]
    dot = functools.partial(jnp.dot, preferred_element_type=jnp.float32)
    u = jax.nn.sigmoid(dot(xg, wua[...]) + dot(hh, wub[...]) + bu[...])
    r = jax.nn.sigmoid(dot(xg, wra[...]) + dot(hh, wrb[...]) + br[...])
    cc = jnp.tanh(dot(xg, wca[...]) + dot(r * hh, wcb[...]) + bc[...])
    out_o[...] = u * hh + (1.0 - u) * cc


def kernel(x, edge_index, edge_weight, h, W1, b1, W2, b2, Wu, bu, Wr, br,
           Wc, bc):
    n, d = x.shape
    e = edge_weight.shape[0]
    f32 = jnp.float32

    rows_pw = -(-e // (N_WORKERS * CH))
    e2 = N_WORKERS * rows_pw * CH
    pad = e2 - e
    src = edge_index[0].astype(jnp.int32)
    dst = edge_index[1].astype(jnp.int32)
    w = edge_weight.astype(f32)
    mask = (1 << (n.bit_length() - 1)) - 1
    pad_idx = jnp.arange(pad, dtype=jnp.int32) & mask
    src3d = jnp.concatenate([src, pad_idx]).reshape(-1, 1, CH)
    dst3d = jnp.concatenate([dst, pad_idx]).reshape(-1, 1, CH)
    w3d = jnp.concatenate([w, jnp.zeros((pad,), f32)]).reshape(-1, 1, CH)

    n_pad = -(-n // (N_SUB * 128)) * (N_SUB * 128)
    zeros1d = np.zeros((n_pad,), np.float32)
    n_acc = -(-n // (N_SUB * 8)) * (N_SUB * 8)
    zeros2d = np.zeros((n_acc, d), np.float32)

    deg_fn = pl.kernel(
        functools.partial(_deg_body, n_pad, rows_pw),
        out_type=[jax.ShapeDtypeStruct((N_CORES * n_pad,), f32)],
        mesh=_MESH,
        scratch_types=[
            pltpu.VMEM_SHARED((n_pad,), f32),
            pltpu.VMEM((rows_pw, 1, CH), jnp.int32),
            pltpu.VMEM((rows_pw, 1, CH), f32),
        ],
        compiler_params=_SC_PARAMS,
    )
    (degp,) = deg_fn(dst3d, w3d, zeros1d)
    deg0 = degp[:n, None]
    deg1 = degp[n_pad:n_pad + n, None]

    conv_fn = pl.kernel(
        functools.partial(_conv_body, n_acc, rows_pw),
        out_type=[jax.ShapeDtypeStruct((N_CORES, n_acc, d), f32)],
        mesh=_MESH,
        scratch_types=(
            [pltpu.VMEM_SHARED((n_acc, d), f32)]
            + [pltpu.VMEM((CH, d), f32) for _ in range(3)]
            + [pltpu.VMEM((CH,), jnp.int32) for _ in range(6)]
            + [pltpu.VMEM((CH,), f32) for _ in range(3)]
            + [pltpu.SemaphoreType.DMA for _ in range(15)]
        ),
        compiler_params=_SC_PARAMS,
    )

    blk = 2000
    grid = (n // blk,)
    full = lambda shape: pl.BlockSpec(shape, lambda i: (0,) * len(shape))
    rowb = lambda shape: pl.BlockSpec(
        shape, lambda i, _l=len(shape): (i,) + (0,) * (_l - 1))
    xw = pl.pallas_call(
        _tc0_body,
        grid=grid,
        in_specs=[rowb((blk, d)), full((d, d))],
        out_specs=rowb((blk, d)),
        out_shape=jax.ShapeDtypeStruct((n, d), f32),
    )(x, W1)
    dinv, y1 = pl.pallas_call(
        _tc1_body,
        grid=grid,
        in_specs=[rowb((blk, 1)), rowb((blk, 1)), rowb((blk, d))],
        out_specs=[rowb((blk, 1)), rowb((blk, d))],
        out_shape=[jax.ShapeDtypeStruct((n, 1), f32),
                   jax.ShapeDtypeStruct((n, d), f32)],
    )(deg0, deg1, xw)

    (acc1,) = conv_fn(y1, src3d, dst3d, w3d, zeros2d)

    accb = pl.BlockSpec((N_CORES, blk, d), lambda i: (0, i, 0))
    gw, y2 = pl.pallas_call(
        _tc2_body,
        grid=grid,
        in_specs=[accb, rowb((blk, 1)), rowb((blk, d)), full((1, d)),
                  full((d, d))],
        out_specs=[rowb((blk, d)), rowb((blk, d))],
        out_shape=[jax.ShapeDtypeStruct((n, d), f32),
                   jax.ShapeDtypeStruct((n, d), f32)],
    )(acc1, dinv, xw, b1.reshape(1, d), W2)

    (acc2,) = conv_fn(y2, src3d, dst3d, w3d, zeros2d)

    hd = Wu.shape[1]
    out = pl.pallas_call(
        _tc3_body,
        grid=grid,
        in_specs=[accb, rowb((blk, 1)), rowb((blk, d)), full((1, d)),
                  rowb((blk, hd)),
                  full((d, hd)), full((hd, hd)), full((1, hd)),
                  full((d, hd)), full((hd, hd)), full((1, hd)),
                  full((d, hd)), full((hd, hd)), full((1, hd))],
        out_specs=rowb((blk, hd)),
        out_shape=jax.ShapeDtypeStruct((n, hd), f32),
    )(acc2, dinv, gw, b2.reshape(1, d), h,
      Wu[:d], Wu[d:], bu.reshape(1, hd),
      Wr[:d], Wr[d:], br.reshape(1, hd),
      Wc[:d], Wc[d:], bc.reshape(1, hd))
    return out


# speedup vs baseline: 1.0359x; 1.0359x over previous
import functools

import jax
import jax.numpy as jnp
import numpy as np
from jax import lax
from jax.experimental import pallas as pl
from jax.experimental.pallas import tpu as pltpu
from jax.experimental.pallas import tpu_sc as plsc

N_CORES = 2
N_SUB = 16
N_WORKERS = N_CORES * N_SUB
CH = 128

_MESH = plsc.VectorSubcoreMesh(
    core_axis_name="c", subcore_axis_name="s",
    num_cores=N_CORES, num_subcores=N_SUB)
_SC_PARAMS = pltpu.CompilerParams(needs_layout_passes=False)


def _deg_body(n_pad, q, rem, e4, wflat, zeros1d, degp, deg_sh, dstv, wv):
    c = lax.axis_index("c")
    s = lax.axis_index("s")
    wid = c * N_SUB + s
    per_tile = n_pad // N_SUB
    pltpu.sync_copy(zeros1d.at[pl.ds(s * per_tile, per_tile)],
                    deg_sh.at[pl.ds(s * per_tile, per_tile)])
    plsc.subcore_barrier()
    base = wid * q + jnp.minimum(wid, rem)
    nch = q + (wid < rem).astype(jnp.int32)
    pltpu.sync_copy(e4.at[1, pl.ds(base, q)], dstv.at[pl.ds(0, q)])
    pltpu.sync_copy(wflat.at[pl.ds(base * CH, q * CH)],
                    wv.at[pl.ds(0, q * CH)])
    if rem:
        @pl.when(nch > q)
        def _():
            pltpu.sync_copy(e4.at[1, pl.ds(base + q, 1)],
                            dstv.at[pl.ds(q, 1)])
            pltpu.sync_copy(wflat.at[pl.ds((base + q) * CH, CH)],
                            wv.at[pl.ds(q * CH, CH)])

    def chunk(i, _):
        @pl.when(i < nch)
        def _():
            pltpu.sync_copy(wv.at[pl.ds(i * CH, CH)],
                            deg_sh.at[dstv.at[i, 0]], add=True)
        return 0

    lax.fori_loop(0, q + (1 if rem else 0), chunk, 0)
    plsc.subcore_barrier()
    pltpu.sync_copy(deg_sh.at[pl.ds(s * per_tile, per_tile)],
                    degp.at[pl.ds(c * n_pad + s * per_tile, per_tile)])


def _conv_body(n_acc, q, rem, y_hbm, e4, wflat, acc_out,
               acc_sh, rows0, rows1, rows2, sb0, sb1, db0, db1, db2,
               wb0, wb1, sg0, sg1, sg2, ss0, ss1, ss2,
               es0, es1, ed0, ed1, ed2, ew0, ew1):
    c = lax.axis_index("c")
    s = lax.axis_index("s")
    wid = c * N_SUB + s
    per_tile = n_acc // N_SUB

    rows = (rows0, rows1, rows2)
    srcb = (sb0, sb1)
    dstb = (db0, db1, db2)
    wb = (wb0, wb1)
    sg = (sg0, sg1, sg2)
    ss = (ss0, ss1, ss2)
    es = (es0, es1)
    ed = (ed0, ed1, ed2)
    ew = (ew0, ew1)

    def zrow(r, _):
        for f in range(8):
            rows0[r, pl.ds(f * 16, 16)] = jnp.zeros((16,), jnp.float32)
        return 0

    lax.fori_loop(0, CH, zrow, 0)
    n_zc, z_rem = divmod(per_tile, CH)
    for k in range(n_zc):
        pltpu.sync_copy(rows0, acc_sh.at[pl.ds(s * per_tile + k * CH,
                                               CH)])
    if z_rem:
        pltpu.sync_copy(rows0.at[pl.ds(0, z_rem)],
                        acc_sh.at[pl.ds(s * per_tile + n_zc * CH,
                                        z_rem)])
    plsc.subcore_barrier()

    base = wid * q + jnp.minimum(wid, rem)
    nch = q + (wid < rem).astype(jnp.int32)

    def load_src(i, k):
        pltpu.async_copy(e4.at[0, i + base, 0], srcb[k], es[k])

    def load_dst(i, k):
        pltpu.async_copy(e4.at[1, i + base, 0], dstb[k], ed[k])

    def load_w(i, k):
        pltpu.async_copy(wflat.at[pl.ds((i + base) * CH, CH)], wb[k],
                         ew[k])

    def scale(i, p3, p2):
        @plsc.parallel_loop(0, CH, unroll=8)
        def body(e):
            ws = plsc.load_gather(wb[p2], [jnp.full((16,), e,
                                                    jnp.int32)])
            for f in range(8):
                sl = pl.ds(f * 16, 16)
                rows[p3][e, sl] = rows[p3][e, sl] * ws

    def step(i, p3, p2):
        r3 = (p3 + 1) % 3
        r2 = 1 - p2

        @pl.when(jnp.logical_and(i >= 2, i < nch + 2))
        def _():
            pltpu.make_async_copy(rows[r3], acc_sh.at[dstb[0]],
                                  ss[r3]).wait()

        @pl.when(i + 1 < nch)
        def _():
            pltpu.make_async_copy(e4.at[0, base, 0], srcb[r2],
                                  es[r2]).wait()
            pltpu.async_copy(y_hbm.at[srcb[r2]], rows[r3], sg[r3])
            pltpu.async_copy(e4.at[1, i + 1 + base, 0], dstb[r3],
                             ed[r3])

        @pl.when(i < nch)
        def _():
            pltpu.make_async_copy(y_hbm.at[srcb[0]], rows[p3],
                                  sg[p3]).wait()

            @pl.when(i + 2 < nch)
            def _():
                load_src(i + 2, p2)
            pltpu.make_async_copy(wflat.at[pl.ds(0, CH)], wb[p2],
                                  ew[p2]).wait()
            scale(i, p3, p2)

            @pl.when(i + 2 < nch)
            def _():
                load_w(i + 2, p2)
            pltpu.make_async_copy(e4.at[1, base, 0], dstb[p3],
                                  ed[p3]).wait()
            pltpu.async_copy(rows[p3], acc_sh.at[dstb[p3]], ss[p3],
                             add=True)

    load_src(0, 0)
    load_w(0, 0)
    load_dst(0, 0)
    load_src(1, 1)
    load_w(1, 1)
    pltpu.make_async_copy(e4.at[0, base, 0], srcb[0], es[0]).wait()
    pltpu.async_copy(y_hbm.at[srcb[0]], rows[0], sg[0])

    max_i = q + (1 if rem else 0) + 2
    n_six = -(-max_i // 6)

    def six(t, _):
        i0 = 6 * t
        for j in range(6):
            step(i0 + j, j % 3, j % 2)
        return 0

    lax.fori_loop(0, n_six, six, 0)

    plsc.subcore_barrier()
    pltpu.sync_copy(acc_sh.at[pl.ds(s * per_tile, per_tile)],
                    acc_out.at[c, pl.ds(s * per_tile, per_tile)])


def _tc0_body(x, w1, xw_o):
    xw_o[...] = jnp.dot(x[...], w1[...],
                        preferred_element_type=jnp.float32)


def _tc1_body(deg0, deg1, xw, dinv_o, y1_o):
    deg = deg0[...] + deg1[...] + 1.0
    dinv = lax.rsqrt(deg)
    dinv_o[...] = dinv
    y1_o[...] = xw[...] * dinv


def _tc2_body(acc, dinv, xw, b1, w2, gw_o, y2_o):
    dv = dinv[...]
    g = acc[0] + acc[1]
    g = jnp.maximum(dv * g + dv * dv * xw[...] + b1[...], 0.0)
    gw = jnp.dot(g, w2[...], preferred_element_type=jnp.float32)
    gw_o[...] = gw
    y2_o[...] = gw * dv


def _tc3_body(acc, dinv, gw, b2, h, wua, wub, bu, wra, wrb, br, wca, wcb,
              bc, out_o):
    dv = dinv[...]
    g = acc[0] + acc[1]
    xg = jax.nn.sigmoid(dv * g + dv * dv * gw[...] + b2[...])
    hh = h[...]
    dot = functools.partial(jnp.dot, preferred_element_type=jnp.float32)
    u = jax.nn.sigmoid(dot(xg, wua[...]) + dot(hh, wub[...]) + bu[...])
    r = jax.nn.sigmoid(dot(xg, wra[...]) + dot(hh, wrb[...]) + br[...])
    cc = jnp.tanh(dot(xg, wca[...]) + dot(r * hh, wcb[...]) + bc[...])
    out_o[...] = u * hh + (1.0 - u) * cc


def kernel(x, edge_index, edge_weight, h, W1, b1, W2, b2, Wu, bu, Wr, br,
           Wc, bc):
    n, d = x.shape
    e = edge_weight.shape[0]
    f32 = jnp.float32
    assert e % CH == 0

    nch_total = e // CH
    q, rem = divmod(nch_total, N_WORKERS)
    e4 = edge_index.astype(jnp.int32).reshape(2, nch_total, 1, CH)
    wflat = edge_weight.astype(f32)

    n_pad = -(-n // (N_SUB * 128)) * (N_SUB * 128)
    zeros1d = np.zeros((n_pad,), np.float32)
    n_acc = -(-n // (N_SUB * 8)) * (N_SUB * 8)

    deg_fn = pl.kernel(
        functools.partial(_deg_body, n_pad, q, rem),
        out_type=[jax.ShapeDtypeStruct((N_CORES * n_pad,), f32)],
        mesh=_MESH,
        scratch_types=[
            pltpu.VMEM_SHARED((n_pad,), f32),
            pltpu.VMEM((q + 1, 1, CH), jnp.int32),
            pltpu.VMEM(((q + 1) * CH,), f32),
        ],
        compiler_params=_SC_PARAMS,
    )
    (degp,) = deg_fn(e4, wflat, zeros1d)
    deg0 = degp[:n, None]
    deg1 = degp[n_pad:n_pad + n, None]

    conv_fn = pl.kernel(
        functools.partial(_conv_body, n_acc, q, rem),
        out_type=[jax.ShapeDtypeStruct((N_CORES, n_acc, d), f32)],
        mesh=_MESH,
        scratch_types=(
            [pltpu.VMEM_SHARED((n_acc, d), f32)]
            + [pltpu.VMEM((CH, d), f32) for _ in range(3)]
            + [pltpu.VMEM((CH,), jnp.int32) for _ in range(5)]
            + [pltpu.VMEM((CH,), f32) for _ in range(2)]
            + [pltpu.SemaphoreType.DMA for _ in range(13)]
        ),
        compiler_params=_SC_PARAMS,
    )

    blk = 2000
    grid = (n // blk,)
    full = lambda shape: pl.BlockSpec(shape, lambda i: (0,) * len(shape))
    rowb = lambda shape: pl.BlockSpec(
        shape, lambda i, _l=len(shape): (i,) + (0,) * (_l - 1))
    xw = pl.pallas_call(
        _tc0_body,
        grid=grid,
        in_specs=[rowb((blk, d)), full((d, d))],
        out_specs=rowb((blk, d)),
        out_shape=jax.ShapeDtypeStruct((n, d), f32),
    )(x, W1)
    dinv, y1 = pl.pallas_call(
        _tc1_body,
        grid=grid,
        in_specs=[rowb((blk, 1)), rowb((blk, 1)), rowb((blk, d))],
        out_specs=[rowb((blk, 1)), rowb((blk, d))],
        out_shape=[jax.ShapeDtypeStruct((n, 1), f32),
                   jax.ShapeDtypeStruct((n, d), f32)],
    )(deg0, deg1, xw)

    (acc1,) = conv_fn(y1, e4, wflat)

    accb = pl.BlockSpec((N_CORES, blk, d), lambda i: (0, i, 0))
    gw, y2 = pl.pallas_call(
        _tc2_body,
        grid=grid,
        in_specs=[accb, rowb((blk, 1)), rowb((blk, d)), full((1, d)),
                  full((d, d))],
        out_specs=[rowb((blk, d)), rowb((blk, d))],
        out_shape=[jax.ShapeDtypeStruct((n, d), f32),
                   jax.ShapeDtypeStruct((n, d), f32)],
    )(acc1, dinv, xw, b1.reshape(1, d), W2)

    (acc2,) = conv_fn(y2, e4, wflat)

    hd = Wu.shape[1]
    out = pl.pallas_call(
        _tc3_body,
        grid=grid,
        in_specs=[accb, rowb((blk, 1)), rowb((blk, d)), full((1, d)),
                  rowb((blk, hd)),
                  full((d, hd)), full((hd, hd)), full((1, hd)),
                  full((d, hd)), full((hd, hd)), full((1, hd)),
                  full((d, hd)), full((hd, hd)), full((1, hd))],
        out_specs=rowb((blk, hd)),
        out_shape=jax.ShapeDtypeStruct((n, hd), f32),
    )(acc2, dinv, gw, b2.reshape(1, d), h,
      Wu[:d], Wu[d:], bu.reshape(1, hd),
      Wr[:d], Wr[d:], br.reshape(1, hd),
      Wc[:d], Wc[d:], bc.reshape(1, hd))
    return out
